# Initial kernel scaffold; baseline (speedup 1.0000x reference)
import jax
import jax.numpy as jnp
from jax.experimental import pallas as pl


def kernel(x, edge_index, W1, b1, W2, b2, W3, b3):
    raise NotImplementedError("write your pallas kernel here")


# speedup vs baseline: 3.0556x; 3.0556x over previous
import functools

import jax
import jax.numpy as jnp
from jax import lax
from jax.experimental import pallas as pl
from jax.experimental.pallas import tpu as pltpu
from jax.experimental.pallas import tpu_sc as plsc

N = 10000
E = 320000
D = 128
NTILES = 32
C = 32768
CH = 6400
B = 128
R = 312
RLAST = 328
NEG_INF = float("-inf")

_MESH = plsc.VectorSubcoreMesh(core_axis_name="c", subcore_axis_name="s")
_SC_PARAMS = pltpu.CompilerParams(needs_layout_passes=False)


def _wid():
    return lax.axis_index("s") * 2 + lax.axis_index("c")


def _lo_of(wid):
    return wid * R


@functools.partial(
    pl.kernel,
    mesh=_MESH,
    out_type=(
        jax.ShapeDtypeStruct((NTILES * C,), jnp.int32),
        jax.ShapeDtypeStruct((NTILES * C,), jnp.int32),
        jax.ShapeDtypeStruct((NTILES * 16,), jnp.int32),
    ),
    scratch_types=[
        pltpu.VMEM((CH,), jnp.int32),
        pltpu.VMEM((CH,), jnp.int32),
        pltpu.VMEM((C,), jnp.int32),
        pltpu.VMEM((C,), jnp.int32),
        pltpu.VMEM((16,), jnp.int32),
    ],
    compiler_params=_SC_PARAMS,
)
def _bin_kernel(dst_hbm, src_hbm, srcl_hbm, ldstl_hbm, cnt_hbm,
                dstc, srcc, srcacc, ldstacc, cntv):
    wid = _wid()
    lo = _lo_of(wid)
    hi = lo + jnp.where(wid == NTILES - 1, RLAST, R)
    lo_v = jnp.broadcast_to(lo, (16,))
    hi_v = jnp.broadcast_to(hi, (16,))
    one_v = jnp.full((16,), 1, jnp.int32)
    cap_v = jnp.full((16,), C, jnp.int32)

    def chunk(c, off):
        pltpu.sync_copy(dst_hbm.at[pl.ds(c * CH, CH)], dstc)
        pltpu.sync_copy(src_hbm.at[pl.ds(c * CH, CH)], srcc)

        def grp(i, off):
            d = dstc[pl.ds(i * 16, 16)]
            s = srcc[pl.ds(i * 16, 16)]
            mask = (d >= lo_v) & (d < hi_v)
            incl = plsc.cumsum(mask.astype(jnp.int32))
            idx = (off + incl) - one_v
            m2 = mask & (idx < cap_v)
            plsc.store_scatter(srcacc, [idx], s, mask=m2)
            plsc.store_scatter(ldstacc, [idx], d - lo_v, mask=m2)
            return off + plsc.all_reduce_population_count(mask)

        return lax.fori_loop(0, CH // 16, grp, off)

    off = lax.fori_loop(0, E // CH, chunk, jnp.zeros((16,), jnp.int32))
    cntv[...] = jnp.minimum(off, cap_v)
    pltpu.sync_copy(srcacc, srcl_hbm.at[pl.ds(wid * C, C)])
    pltpu.sync_copy(ldstacc, ldstl_hbm.at[pl.ds(wid * C, C)])
    pltpu.sync_copy(cntv, cnt_hbm.at[pl.ds(wid * 16, 16)])


@functools.partial(
    pl.kernel,
    mesh=_MESH,
    out_type=jax.ShapeDtypeStruct((N, D), jnp.float32),
    scratch_types=[
        pltpu.VMEM((RLAST, D), jnp.float32),
        pltpu.VMEM((B, D), jnp.float32),
        pltpu.VMEM((B,), jnp.int32),
        pltpu.VMEM((B + 16,), jnp.int32),
        pltpu.VMEM((16,), jnp.int32),
        pltpu.VMEM((8, D), jnp.float32),
        pltpu.SemaphoreType.DMA,
    ],
    compiler_params=_SC_PARAMS,
)
def _emax_kernel(u_hbm, v_hbm, srcl_hbm, ldstl_hbm, cnt_hbm, out_hbm,
                 acc, vrow, srcc, ldstc, cntv, ubuf, sem):
    wid = _wid()
    lo = _lo_of(wid)

    neg = jnp.full((16,), NEG_INF, jnp.float32)

    def init(r, _):
        for k in range(8):
            acc[r, pl.ds(k * 16, 16)] = neg
        return 0

    lax.fori_loop(0, RLAST, init, 0)

    pltpu.sync_copy(cnt_hbm.at[pl.ds(wid * 16, 16)], cntv)
    cnt = cntv[pl.ds(0, 16)][0]
    nb = (cnt + B - 1) // B

    def batch(g, _):
        pltpu.sync_copy(srcl_hbm.at[pl.ds(wid * C + g * B, B)], srcc)
        zero_v = jnp.zeros((16,), jnp.int32)
        nmax_v = jnp.full((16,), N - 1, jnp.int32)
        for i in range(B // 16):
            s = srcc[pl.ds(i * 16, 16)]
            srcc[pl.ds(i * 16, 16)] = jnp.minimum(jnp.maximum(s, zero_v),
                                                  nmax_v)
        pltpu.async_copy(v_hbm.at[srcc], vrow, sem).wait()
        pltpu.sync_copy(ldstl_hbm.at[pl.ds(wid * C + g * B, B)],
                        ldstc.at[pl.ds(0, B)])
        m = jnp.minimum(B, cnt - g * B)

        def edge(e, _):
            ld = ldstc[pl.ds(e, 16)][0]
            for k in range(8):
                a = acc[ld, pl.ds(k * 16, 16)]
                v = vrow[e, pl.ds(k * 16, 16)]
                acc[ld, pl.ds(k * 16, 16)] = jnp.maximum(a, v)
            return 0

        lax.fori_loop(0, m, edge, 0)
        return 0

    lax.fori_loop(0, nb, batch, 0)

    zero_f = jnp.zeros((16,), jnp.float32)

    def outgrp(g, _):
        pltpu.sync_copy(u_hbm.at[pl.ds(lo + g * 8, 8)], ubuf)
        for r in range(8):
            for k in range(8):
                a = acc[g * 8 + r, pl.ds(k * 16, 16)]
                u = ubuf[r, pl.ds(k * 16, 16)]
                acc[g * 8 + r, pl.ds(k * 16, 16)] = jnp.where(
                    a == neg, zero_f, a + u)
        return 0

    ngrp = jnp.where(wid == NTILES - 1, RLAST // 8, R // 8)
    lax.fori_loop(0, ngrp, outgrp, 0)

    @pl.when(wid < NTILES - 1)
    def _():
        pltpu.sync_copy(acc.at[pl.ds(0, R)], out_hbm.at[pl.ds(lo, R)])

    @pl.when(wid == NTILES - 1)
    def _():
        pltpu.sync_copy(acc.at[pl.ds(0, RLAST)], out_hbm.at[pl.ds(lo, RLAST)])


def _proj_body(h_ref, w_ref, b_ref, u_ref, v_ref, *, relu):
    h = h_ref[...]
    if relu:
        h = jnp.maximum(h, 0.0)
    out = jnp.dot(h, w_ref[...], preferred_element_type=jnp.float32)
    u_ref[...] = out[:, :D] + b_ref[...]
    v_ref[...] = out[:, D:]


def _proj(h, wcat, b, relu):
    return pl.pallas_call(
        functools.partial(_proj_body, relu=relu),
        grid=(10,),
        in_specs=[
            pl.BlockSpec((1000, D), lambda i: (i, 0)),
            pl.BlockSpec((D, 2 * D), lambda i: (0, 0)),
            pl.BlockSpec((1, D), lambda i: (0, 0)),
        ],
        out_specs=[
            pl.BlockSpec((1000, D), lambda i: (i, 0)),
            pl.BlockSpec((1000, D), lambda i: (i, 0)),
        ],
        out_shape=[
            jax.ShapeDtypeStruct((N, D), jnp.float32),
            jax.ShapeDtypeStruct((N, D), jnp.float32),
        ],
    )(h, wcat, b)


def _wcat(W):
    wa, wb = W[:D], W[D:]
    return jnp.concatenate([wa - wb, wb], axis=1)


def kernel(x, edge_index, W1, b1, W2, b2, W3, b3):
    src = edge_index[0]
    dst = edge_index[1]
    srcl, ldstl, cnts = _bin_kernel(dst, src)

    u, v = _proj(x, _wcat(W1), b1.reshape(1, D), relu=False)
    h = _emax_kernel(u, v, srcl, ldstl, cnts)
    u, v = _proj(h, _wcat(W2), b2.reshape(1, D), relu=True)
    h = _emax_kernel(u, v, srcl, ldstl, cnts)
    u, v = _proj(h, _wcat(W3), b3.reshape(1, D), relu=True)
    return _emax_kernel(u, v, srcl, ldstl, cnts)


# speedup vs baseline: 3.5720x; 1.1690x over previous
import functools

import jax
import jax.numpy as jnp
from jax import lax
from jax.experimental import pallas as pl
from jax.experimental.pallas import tpu as pltpu
from jax.experimental.pallas import tpu_sc as plsc

N = 10000
E = 320000
D = 128
NTILES = 32
C = 32768
CH = 6400
B = 128
R = 312
RLAST = 328
NEG_INF = float("-inf")

_MESH = plsc.VectorSubcoreMesh(core_axis_name="c", subcore_axis_name="s")
_SC_PARAMS = pltpu.CompilerParams(needs_layout_passes=False)


def _wid():
    return lax.axis_index("s") * 2 + lax.axis_index("c")


def _lo_of(wid):
    return wid * R


@functools.partial(
    pl.kernel,
    mesh=_MESH,
    out_type=(
        jax.ShapeDtypeStruct((NTILES * C,), jnp.int32),
        jax.ShapeDtypeStruct((NTILES * C,), jnp.int32),
        jax.ShapeDtypeStruct((NTILES * 16,), jnp.int32),
    ),
    scratch_types=[
        pltpu.VMEM((CH,), jnp.int32),
        pltpu.VMEM((CH,), jnp.int32),
        pltpu.VMEM((C,), jnp.int32),
        pltpu.VMEM((C,), jnp.int32),
        pltpu.VMEM((16,), jnp.int32),
    ],
    compiler_params=_SC_PARAMS,
)
def _bin_kernel(dst_hbm, src_hbm, srcl_hbm, ldstl_hbm, cnt_hbm,
                dstc, srcc, srcacc, ldstacc, cntv):
    wid = _wid()
    lo = _lo_of(wid)
    hi = lo + jnp.where(wid == NTILES - 1, RLAST, R)
    lo_v = jnp.broadcast_to(lo, (16,))
    hi_v = jnp.broadcast_to(hi, (16,))
    one_v = jnp.full((16,), 1, jnp.int32)
    cap_v = jnp.full((16,), C, jnp.int32)

    def chunk(c, off):
        pltpu.sync_copy(dst_hbm.at[pl.ds(c * CH, CH)], dstc)
        pltpu.sync_copy(src_hbm.at[pl.ds(c * CH, CH)], srcc)

        def grp(i, off):
            d = dstc[pl.ds(i * 16, 16)]
            s = srcc[pl.ds(i * 16, 16)]
            mask = (d >= lo_v) & (d < hi_v)
            incl = plsc.cumsum(mask.astype(jnp.int32))
            idx = (off + incl) - one_v
            m2 = mask & (idx < cap_v)
            plsc.store_scatter(srcacc, [idx], s, mask=m2)
            plsc.store_scatter(ldstacc, [idx], d - lo_v, mask=m2)
            return off + plsc.all_reduce_population_count(mask)

        return lax.fori_loop(0, CH // 16, grp, off)

    off = lax.fori_loop(0, E // CH, chunk, jnp.zeros((16,), jnp.int32))
    cntv[...] = jnp.minimum(off, cap_v)
    pltpu.sync_copy(srcacc, srcl_hbm.at[pl.ds(wid * C, C)])
    pltpu.sync_copy(ldstacc, ldstl_hbm.at[pl.ds(wid * C, C)])
    pltpu.sync_copy(cntv, cnt_hbm.at[pl.ds(wid * 16, 16)])


@functools.partial(
    pl.kernel,
    mesh=_MESH,
    out_type=jax.ShapeDtypeStruct((N, D), jnp.float32),
    scratch_types=[
        pltpu.VMEM((RLAST + 8, D), jnp.float32),
        pltpu.VMEM((B, D), jnp.float32),
        pltpu.VMEM((B,), jnp.int32),
        pltpu.VMEM((B + 16,), jnp.int32),
        pltpu.VMEM((16,), jnp.int32),
        pltpu.VMEM((8, D), jnp.float32),
        pltpu.SemaphoreType.DMA,
    ],
    compiler_params=_SC_PARAMS,
)
def _emax_kernel(u_hbm, v_hbm, srcl_hbm, ldstl_hbm, cnt_hbm, out_hbm,
                 acc, vrow, srcc, ldstc, cntv, ubuf, sem):
    wid = _wid()
    lo = _lo_of(wid)

    neg = jnp.full((16,), NEG_INF, jnp.float32)

    def init(r, _):
        for k in range(8):
            acc[r, pl.ds(k * 16, 16)] = neg
        return 0

    lax.fori_loop(0, RLAST, init, 0)

    pltpu.sync_copy(cnt_hbm.at[pl.ds(wid * 16, 16)], cntv)
    cnt = cntv[pl.ds(0, 16)][0]
    nb = (cnt + B - 1) // B

    def batch(g, _):
        pltpu.sync_copy(srcl_hbm.at[pl.ds(wid * C + g * B, B)], srcc)
        zero_v = jnp.zeros((16,), jnp.int32)
        nmax_v = jnp.full((16,), N - 1, jnp.int32)
        for i in range(B // 16):
            s = srcc[pl.ds(i * 16, 16)]
            srcc[pl.ds(i * 16, 16)] = jnp.minimum(jnp.maximum(s, zero_v),
                                                  nmax_v)
        pltpu.async_copy(v_hbm.at[srcc], vrow, sem).wait()
        pltpu.sync_copy(ldstl_hbm.at[pl.ds(wid * C + g * B, B)],
                        ldstc.at[pl.ds(0, B)])
        m = jnp.minimum(B, cnt - g * B)
        m_v = jnp.broadcast_to(m, (16,))
        lane = lax.iota(jnp.int32, 16)
        trash = jnp.full((16,), RLAST, jnp.int32)

        def blk(q, _):
            ld16 = ldstc[pl.ds(q * 16, 16)]
            base = jnp.broadcast_to(q * 16, (16,))
            ld16 = jnp.where((base + lane) < m_v, ld16, trash)
            for j in range(16):
                ld = ld16[j]
                e = q * 16 + j
                for k in range(8):
                    a = acc[ld, pl.ds(k * 16, 16)]
                    v = vrow[e, pl.ds(k * 16, 16)]
                    acc[ld, pl.ds(k * 16, 16)] = jnp.maximum(a, v)
            return 0

        lax.fori_loop(0, (m + 15) // 16, blk, 0)
        return 0

    lax.fori_loop(0, nb, batch, 0)

    zero_f = jnp.zeros((16,), jnp.float32)

    def outgrp(g, _):
        pltpu.sync_copy(u_hbm.at[pl.ds(lo + g * 8, 8)], ubuf)
        for r in range(8):
            for k in range(8):
                a = acc[g * 8 + r, pl.ds(k * 16, 16)]
                u = ubuf[r, pl.ds(k * 16, 16)]
                acc[g * 8 + r, pl.ds(k * 16, 16)] = jnp.where(
                    a == neg, zero_f, a + u)
        return 0

    ngrp = jnp.where(wid == NTILES - 1, RLAST // 8, R // 8)
    lax.fori_loop(0, ngrp, outgrp, 0)

    @pl.when(wid < NTILES - 1)
    def _():
        pltpu.sync_copy(acc.at[pl.ds(0, R)], out_hbm.at[pl.ds(lo, R)])

    @pl.when(wid == NTILES - 1)
    def _():
        pltpu.sync_copy(acc.at[pl.ds(0, RLAST)], out_hbm.at[pl.ds(lo, RLAST)])


def _proj_body(h_ref, w_ref, b_ref, u_ref, v_ref, *, relu):
    h = h_ref[...]
    if relu:
        h = jnp.maximum(h, 0.0)
    out = jnp.dot(h, w_ref[...], preferred_element_type=jnp.float32)
    u_ref[...] = out[:, :D] + b_ref[...]
    v_ref[...] = out[:, D:]


def _proj(h, wcat, b, relu):
    return pl.pallas_call(
        functools.partial(_proj_body, relu=relu),
        grid=(10,),
        in_specs=[
            pl.BlockSpec((1000, D), lambda i: (i, 0)),
            pl.BlockSpec((D, 2 * D), lambda i: (0, 0)),
            pl.BlockSpec((1, D), lambda i: (0, 0)),
        ],
        out_specs=[
            pl.BlockSpec((1000, D), lambda i: (i, 0)),
            pl.BlockSpec((1000, D), lambda i: (i, 0)),
        ],
        out_shape=[
            jax.ShapeDtypeStruct((N, D), jnp.float32),
            jax.ShapeDtypeStruct((N, D), jnp.float32),
        ],
    )(h, wcat, b)


def _wcat(W):
    wa, wb = W[:D], W[D:]
    return jnp.concatenate([wa - wb, wb], axis=1)


def kernel(x, edge_index, W1, b1, W2, b2, W3, b3):
    src = edge_index[0]
    dst = edge_index[1]
    srcl, ldstl, cnts = _bin_kernel(dst, src)

    u, v = _proj(x, _wcat(W1), b1.reshape(1, D), relu=False)
    h = _emax_kernel(u, v, srcl, ldstl, cnts)
    u, v = _proj(h, _wcat(W2), b2.reshape(1, D), relu=True)
    h = _emax_kernel(u, v, srcl, ldstl, cnts)
    u, v = _proj(h, _wcat(W3), b3.reshape(1, D), relu=True)
    return _emax_kernel(u, v, srcl, ldstl, cnts)
